# baseline (speedup 1.0000x reference)
import jax
import jax.numpy as jnp
from jax.experimental import pallas as pl
from jax.experimental.pallas import tpu as pltpu

_NBUF = 8
_LAG = 4
_BR = 512


def _ring_copy(x_hbm, o_hbm, buf, in_sems, out_sems):
    rows = x_hbm.shape[0]
    n = rows // _BR

    def in_copy(i):
        return pltpu.make_async_copy(
            x_hbm.at[pl.ds(i * _BR, _BR)], buf.at[i % _NBUF], in_sems.at[i % _NBUF])

    def out_copy(i):
        return pltpu.make_async_copy(
            buf.at[i % _NBUF], o_hbm.at[pl.ds(i * _BR, _BR)], out_sems.at[i % _NBUF])

    waited = set()
    for i in range(_NBUF):
        in_copy(i).start()
    for i in range(n):
        in_copy(i).wait()
        out_copy(i).start()
        t = i - _LAG
        if 0 <= t and t + _NBUF < n:
            out_copy(t).wait()
            waited.add(t)
            in_copy(t + _NBUF).start()
    for t in range(n):
        if t not in waited:
            out_copy(t).wait()


def kernel(data, partitions):
    del partitions
    rows, cols = data.shape
    return pl.pallas_call(
        _ring_copy,
        in_specs=[pl.BlockSpec(memory_space=pl.ANY)],
        out_specs=pl.BlockSpec(memory_space=pl.ANY),
        scratch_shapes=[
            pltpu.VMEM((_NBUF, _BR, cols), data.dtype),
            pltpu.SemaphoreType.DMA((_NBUF,)),
            pltpu.SemaphoreType.DMA((_NBUF,)),
        ],
        out_shape=jax.ShapeDtypeStruct((rows, cols), data.dtype),
    )(data)
